# Initial kernel scaffold; baseline (speedup 1.0000x reference)
import jax
import jax.numpy as jnp
from jax.experimental import pallas as pl


def kernel(x, edge_index, batch, W1, b1, W2, b2, W3, b3, W4, b4, lin1_W, lin1_b, lin2_W, lin2_b):
    raise NotImplementedError("write your pallas kernel here")


# speedup vs baseline: 7.3374x; 7.3374x over previous
import functools

import jax
import jax.numpy as jnp
from jax import lax
from jax.experimental import pallas as pl
from jax.experimental.pallas import tpu as pltpu
from jax.experimental.pallas import tpu_sc as plsc

N = 10000
E = 320000
H = 128
NUM_GRAPHS = 64

NC = 2
NS = 16
NPC = N // NC
EPT = E // NS
CHUNK = 128
NCHUNK = -(-EPT // CHUNK)
EPT_PAD = NCHUNK * CHUNK

ACC_ROWS = 5120
ROWS_PT = ACC_ROWS // NS

DEG_PAD = 16384
DEG_EPT = DEG_PAD // NS
DEG_OR = DEG_PAD // 128

RB = 200
GRID = N // RB
BPP = NPC // RB

_vmesh = plsc.VectorSubcoreMesh(core_axis_name="c", subcore_axis_name="s")


@functools.partial(
    pl.kernel,
    out_type=jax.ShapeDtypeStruct((DEG_OR, 128), jnp.float32),
    mesh=_vmesh,
    scratch_types=[
        pltpu.VMEM((NCHUNK, CHUNK), jnp.int32),
        pltpu.VMEM((CHUNK,), jnp.float32),
        pltpu.VMEM((DEG_EPT,), jnp.float32),
        pltpu.VMEM((DEG_EPT // 128, 128), jnp.float32),
        pltpu.VMEM_SHARED((DEG_PAD,), jnp.float32),
    ],
)
def _sc_deg(dst_hbm, out_hbm, dst_v, ones_v, bounce_v, out2_v, acc_sh):
    c = lax.axis_index("c")
    s = lax.axis_index("s")
    pltpu.sync_copy(dst_hbm.at[s], dst_v)

    one16 = jnp.ones((16,), jnp.float32)
    zero16 = jnp.zeros((16,), jnp.float32)
    for l in range(CHUNK // 16):
        ones_v[pl.ds(16 * l, 16)] = one16
    for l in range(DEG_EPT // 16):
        bounce_v[pl.ds(16 * l, 16)] = zero16
    pltpu.sync_copy(bounce_v, acc_sh.at[pl.ds(s * DEG_EPT, DEG_EPT)])
    plsc.subcore_barrier()

    def body(j, carry):
        pltpu.sync_copy(ones_v, acc_sh.at[dst_v.at[j]], add=True)
        return carry

    lax.fori_loop(0, NCHUNK, body, 0)
    plsc.subcore_barrier()

    @pl.when(c == 0)
    def _writeout():
        pltpu.sync_copy(acc_sh.at[pl.ds(s * DEG_EPT, DEG_EPT)], bounce_v)
        for r in range(DEG_EPT // 128):
            for l in range(8):
                out2_v[r, pl.ds(16 * l, 16)] = bounce_v[pl.ds((r * 8 + l) * 16, 16)]
        pltpu.sync_copy(out2_v, out_hbm.at[pl.ds(s * (DEG_EPT // 128), DEG_EPT // 128)])


@functools.partial(
    pl.kernel,
    out_type=jax.ShapeDtypeStruct((NC, ACC_ROWS, H), jnp.float32),
    mesh=_vmesh,
    scratch_types=[
        pltpu.VMEM((NCHUNK, CHUNK), jnp.int32),
        pltpu.VMEM((NCHUNK, CHUNK), jnp.int32),
        pltpu.VMEM((CHUNK, H), jnp.float32),
        pltpu.VMEM((CHUNK, H), jnp.float32),
        pltpu.VMEM_SHARED((ACC_ROWS, H), jnp.float32),
        pltpu.SemaphoreType.DMA,
    ],
)
def _sc_scatter(u_hbm, src_hbm, dstloc_hbm, zeros2_hbm, out_hbm,
                src_v, dst_v, rows_v, zero_v, acc_sh, sem):
    c = lax.axis_index("c")
    s = lax.axis_index("s")
    pltpu.sync_copy(src_hbm.at[s], src_v)
    pltpu.sync_copy(dstloc_hbm.at[c].at[s], dst_v)
    pltpu.sync_copy(zeros2_hbm, zero_v)

    z0 = s * ROWS_PT
    pltpu.sync_copy(zero_v, acc_sh.at[pl.ds(z0, CHUNK)])
    pltpu.sync_copy(zero_v, acc_sh.at[pl.ds(z0 + CHUNK, CHUNK)])
    pltpu.sync_copy(zero_v.at[pl.ds(0, ROWS_PT - 2 * CHUNK)],
                    acc_sh.at[pl.ds(z0 + 2 * CHUNK, ROWS_PT - 2 * CHUNK)])
    plsc.subcore_barrier()

    def body(j, carry):
        pltpu.async_copy(u_hbm.at[src_v.at[j]], rows_v, sem).wait()
        pltpu.sync_copy(rows_v, acc_sh.at[dst_v.at[j]], add=True)
        return carry

    lax.fori_loop(0, NCHUNK, body, 0)
    plsc.subcore_barrier()

    for k in range(3):
        sz = CHUNK if k < 2 else ROWS_PT - 2 * CHUNK
        pltpu.sync_copy(acc_sh.at[pl.ds(z0 + k * CHUNK, sz)],
                        zero_v.at[pl.ds(0, sz)])
        pltpu.sync_copy(zero_v.at[pl.ds(0, sz)],
                        out_hbm.at[c].at[pl.ds(z0 + k * CHUNK, sz)])


def _acc_map(i):
    return (i // BPP, i % BPP, 0)


def _dinv_of(deg_ref):
    return lax.rsqrt(deg_ref[...] + 1.0)


def _tc_first_body(deg_ref, x_ref, w_ref, u_ref):
    dinv = _dinv_of(deg_ref)
    z = jnp.dot(x_ref[...], w_ref[...], preferred_element_type=jnp.float32)
    u_ref[...] = z * dinv


def _tc_first(deg2, x, W1):
    return pl.pallas_call(
        _tc_first_body,
        grid=(GRID,),
        in_specs=[
            pl.BlockSpec((RB, 1), lambda i: (i, 0)),
            pl.BlockSpec((RB, H), lambda i: (i, 0)),
            pl.BlockSpec((H, H), lambda i: (0, 0)),
        ],
        out_specs=pl.BlockSpec((RB, H), lambda i: (i, 0)),
        out_shape=jax.ShapeDtypeStruct((N, H), jnp.float32),
    )(deg2, x, W1)


def _relu_conv(acc_ref, u_ref, deg_ref, b_ref):
    dinv = _dinv_of(deg_ref)
    tot = acc_ref[0] + u_ref[...]
    return jnp.maximum(tot * dinv + b_ref[...], 0.0), dinv


def _tc_mid_body(acc_ref, u_ref, deg_ref, b_ref, w_ref, o_ref):
    h, dinv = _relu_conv(acc_ref, u_ref, deg_ref, b_ref)
    z = jnp.dot(h, w_ref[...], preferred_element_type=jnp.float32)
    o_ref[...] = z * dinv


def _tc_mid(acc, u, deg2, b2d, W):
    return pl.pallas_call(
        _tc_mid_body,
        grid=(GRID,),
        in_specs=[
            pl.BlockSpec((1, RB, H), _acc_map),
            pl.BlockSpec((RB, H), lambda i: (i, 0)),
            pl.BlockSpec((RB, 1), lambda i: (i, 0)),
            pl.BlockSpec((1, H), lambda i: (0, 0)),
            pl.BlockSpec((H, H), lambda i: (0, 0)),
        ],
        out_specs=pl.BlockSpec((RB, H), lambda i: (i, 0)),
        out_shape=jax.ShapeDtypeStruct((N, H), jnp.float32),
    )(acc, u, deg2, b2d, W)


def _tc_final_body(acc_ref, u_ref, deg_ref, b_ref, batch_ref,
                   w1_ref, b1_ref, w2_ref, b2_ref, out_ref, sums_s, cnt_s):
    i = pl.program_id(0)

    @pl.when(i == 0)
    def _init():
        sums_s[...] = jnp.zeros_like(sums_s)
        cnt_s[...] = jnp.zeros_like(cnt_s)

    h, _ = _relu_conv(acc_ref, u_ref, deg_ref, b_ref)
    gids = batch_ref[...]
    onehot = (gids == lax.broadcasted_iota(jnp.int32, (1, NUM_GRAPHS), 1)
              ).astype(jnp.float32)
    sums_s[...] += lax.dot_general(onehot, h, (((0,), (0,)), ((), ())),
                                   preferred_element_type=jnp.float32)
    cnt_s[...] += jnp.sum(onehot, axis=0)[:, None]

    @pl.when(i == GRID - 1)
    def _finish():
        pooled = sums_s[...] / jnp.maximum(cnt_s[...], 1.0)
        g = jnp.maximum(
            jnp.dot(pooled, w1_ref[...], preferred_element_type=jnp.float32)
            + b1_ref[...], 0.0)
        out_ref[...] = (
            jnp.dot(g, w2_ref[...], preferred_element_type=jnp.float32)
            + b2_ref[...])


def _tc_final(acc, u, deg2, b2d, batch2d, lin1_W, lin1_b2d, lin2_W, lin2_b2d):
    return pl.pallas_call(
        _tc_final_body,
        grid=(GRID,),
        in_specs=[
            pl.BlockSpec((1, RB, H), _acc_map),
            pl.BlockSpec((RB, H), lambda i: (i, 0)),
            pl.BlockSpec((RB, 1), lambda i: (i, 0)),
            pl.BlockSpec((1, H), lambda i: (0, 0)),
            pl.BlockSpec((RB, 1), lambda i: (i, 0)),
            pl.BlockSpec((H, H), lambda i: (0, 0)),
            pl.BlockSpec((1, H), lambda i: (0, 0)),
            pl.BlockSpec((H, 1), lambda i: (0, 0)),
            pl.BlockSpec((1, 1), lambda i: (0, 0)),
        ],
        out_specs=pl.BlockSpec((NUM_GRAPHS, 1), lambda i: (0, 0)),
        out_shape=jax.ShapeDtypeStruct((NUM_GRAPHS, 1), jnp.float32),
        scratch_shapes=[
            pltpu.VMEM((NUM_GRAPHS, H), jnp.float32),
            pltpu.VMEM((NUM_GRAPHS, 1), jnp.float32),
        ],
    )(acc, u, deg2, b2d, batch2d, lin1_W, lin1_b2d, lin2_W, lin2_b2d)


def kernel(x, edge_index, batch, W1, b1, W2, b2, W3, b3, W4, b4,
           lin1_W, lin1_b, lin2_W, lin2_b):
    src = edge_index[0].reshape(NS, EPT)
    dst = edge_index[1].reshape(NS, EPT)
    pad = ((0, 0), (0, EPT_PAD - EPT))
    src3 = jnp.pad(src, pad, constant_values=0).reshape(NS, NCHUNK, CHUNK)
    dst_p = jnp.pad(dst, pad, constant_values=N)
    dst0 = jnp.where(dst_p < NPC, dst_p, NPC)
    dst1 = jnp.where(dst_p >= NPC, jnp.minimum(dst_p - NPC, NPC), NPC)
    dstloc = jnp.stack([dst0, dst1]).reshape(NC, NS, NCHUNK, CHUNK)
    dst3 = jnp.pad(dst, pad, constant_values=N).reshape(NS, NCHUNK, CHUNK)

    zeros2 = jnp.zeros((CHUNK, H), jnp.float32)

    deg = _sc_deg(dst3)
    deg2 = deg.reshape(DEG_PAD)[:N].reshape(N, 1)

    u = _tc_first(deg2, x, W1)
    acc = _sc_scatter(u, src3, dstloc, zeros2)
    u = _tc_mid(acc, u, deg2, b1.reshape(1, H), W2)
    acc = _sc_scatter(u, src3, dstloc, zeros2)
    u = _tc_mid(acc, u, deg2, b2.reshape(1, H), W3)
    acc = _sc_scatter(u, src3, dstloc, zeros2)
    u = _tc_mid(acc, u, deg2, b3.reshape(1, H), W4)
    acc = _sc_scatter(u, src3, dstloc, zeros2)

    return _tc_final(acc, u, deg2, b4.reshape(1, H), batch.reshape(N, 1),
                     lin1_W, lin1_b.reshape(1, H), lin2_W,
                     lin2_b.reshape(1, 1))
